# Initial kernel scaffold; baseline (speedup 1.0000x reference)
import jax
import jax.numpy as jnp
from jax.experimental import pallas as pl


def kernel(x, edge_index, W0, b0, Wmid, bmid, Wout, bout):
    raise NotImplementedError("write your pallas kernel here")


# speedup vs baseline: 69.9002x; 69.9002x over previous
import jax
import jax.numpy as jnp
from jax import lax
from jax.experimental import pallas as pl
from jax.experimental.pallas import tpu as pltpu
from jax.experimental.pallas import tpu_sc as plsc

N = 50000
E = 800000
L_MID = 10
NLAYERS = 12

NTILES = 16
NP = 50176
NS = NP // NTILES
NVEC = NS // 16
NRVEC = N // 16
EP = 819200
ET = EP // NTILES
CH = 2048
NCH = ET // CH
ED = E // NTILES
DCH = 2000
NDCH = ED // DCH


def _exp_vec(bits_vec):
  return ((bits_vec >> 23) & 255) - 127 + 1


def _body(x_hbm, s_hbm, d_hbm, sc_hbm, out_hbm,
          wbuf, zbuf, mwa, mwb, dmx,
          wfv, dvv, wiv, tmp, sidx, didx, vals,
          dones, didx2, scv, mrd, mrdi, mxv, dxv, mdv):
  sid = lax.axis_index("s")
  nb = sid * NS
  nsl = pl.ds(nb, NS)

  pltpu.sync_copy(sc_hbm, scv)
  svec = scv[12]

  def zero_fn(i, _):
    wiv[pl.ds(i * 16, 16)] = jnp.zeros((16,), jnp.int32)
    return _
  lax.fori_loop(0, NVEC, zero_fn, None)
  pltpu.sync_copy(wiv, zbuf.at[nsl])

  def ones_fn(i, _):
    dones[pl.ds(i * 16, 16)] = jnp.full((16,), 1, jnp.int32)
    return _
  lax.fori_loop(0, DCH // 16, ones_fn, None)

  plsc.subcore_barrier()

  def deg_fn(k, _):
    eb = sid * ED + k * DCH
    pltpu.sync_copy(d_hbm.at[pl.ds(eb, DCH)], didx2)
    pltpu.sync_copy(dones, zbuf.at[didx2], add=True)
    return _
  lax.fori_loop(0, NDCH, deg_fn, None)

  plsc.subcore_barrier()

  pltpu.sync_copy(zbuf.at[nsl], wiv)
  dxv[...] = jnp.zeros((16,), jnp.int32)

  def rsqrt_fn(i, _):
    di = wiv[pl.ds(i * 16, 16)]
    dxv[...] = jnp.maximum(dxv[...], di)
    xv = (di + 1).astype(jnp.float32)
    ii = lax.bitcast_convert_type(xv, jnp.int32)
    ii = jnp.int32(0x5F3759DF) - lax.shift_right_logical(ii, 1)
    y = lax.bitcast_convert_type(ii, jnp.float32)
    y = y * (1.5 - 0.5 * xv * y * y)
    y = y * (1.5 - 0.5 * xv * y * y)
    y = y * (1.5 - 0.5 * xv * y * y)
    dvv[pl.ds(i * 16, 16)] = y
    return _
  lax.fori_loop(0, NVEC, rsqrt_fn, None)
  pltpu.sync_copy(dxv, dmx.at[pl.ds(sid * 16, 16)])

  pltpu.sync_copy(x_hbm.at[nsl], tmp)
  mxv[...] = jnp.zeros((16,), jnp.float32)

  def w0_fn(i, _):
    w = tmp[pl.ds(i * 16, 16)] * svec * dvv[pl.ds(i * 16, 16)]
    wfv[pl.ds(i * 16, 16)] = w
    mxv[...] = jnp.maximum(mxv[...], jnp.abs(w))
    return _
  lax.fori_loop(0, NVEC, w0_fn, None)
  pltpu.sync_copy(mxv, mwa.at[pl.ds(sid * 16, 16)])

  plsc.subcore_barrier()

  pltpu.sync_copy(dmx, mrdi)
  m = mrdi[pl.ds(0, 16)]
  for r in range(1, NTILES):
    m = jnp.maximum(m, mrdi[pl.ds(r * 16, 16)])
  mdscal = m[0]
  for l in range(1, 16):
    mdscal = jnp.maximum(mdscal, m[l])
  mdscal = mdscal + 1
  mdfv = jnp.full((16,), mdscal, jnp.int32).astype(jnp.float32)
  mdv[...] = _exp_vec(lax.bitcast_convert_type(mdfv, jnp.int32))

  for app in range(NLAYERS):
    mw_in = mwa if app % 2 == 0 else mwb
    mw_out = mwb if app % 2 == 0 else mwa
    bvec = scv[app]

    if app > 0:
      plsc.subcore_barrier()

    pltpu.sync_copy(mw_in, mrd)
    g = mrd[pl.ds(0, 16)]
    for r in range(1, NTILES):
      g = jnp.maximum(g, mrd[pl.ds(r * 16, 16)])
    gs = g[0]
    for l in range(1, 16):
      gs = jnp.maximum(gs, g[l])
    gs = jnp.maximum(gs, jnp.float32(1e-30))
    gv = jnp.full((16,), gs, jnp.float32)
    sexp = 30 - mdv[...] - _exp_vec(lax.bitcast_convert_type(gv, jnp.int32))
    svec_s = lax.bitcast_convert_type(lax.shift_left(sexp + 127, 23),
                                      jnp.float32)
    svec_inv = lax.bitcast_convert_type(lax.shift_left(127 - sexp, 23),
                                        jnp.float32)

    def cvt_fn(i, _):
      wiv[pl.ds(i * 16, 16)] = (
          wfv[pl.ds(i * 16, 16)] * svec_s).astype(jnp.int32)
      return _
    lax.fori_loop(0, NVEC, cvt_fn, None)
    pltpu.sync_copy(wiv, wbuf.at[nsl])
    pltpu.sync_copy(wiv, zbuf.at[nsl])

    plsc.subcore_barrier()

    def edge_fn(k, _):
      eb = sid * ET + k * CH
      pltpu.sync_copy(s_hbm.at[pl.ds(eb, CH)], sidx)
      pltpu.sync_copy(d_hbm.at[pl.ds(eb, CH)], didx)
      pltpu.sync_copy(wbuf.at[sidx], vals)
      pltpu.sync_copy(vals, zbuf.at[didx], add=True)
      return _
    lax.fori_loop(0, NCH, edge_fn, None)

    plsc.subcore_barrier()

    pltpu.sync_copy(zbuf.at[nsl], wiv)
    if app < NLAYERS - 1:
      mxv[...] = jnp.zeros((16,), jnp.float32)

      def y_fn(i, _):
        dv = dvv[pl.ds(i * 16, 16)]
        zf = wiv[pl.ds(i * 16, 16)].astype(jnp.float32)
        yf = zf * svec_inv * dv + bvec
        pad_ok = jnp.where(sid * NVEC + i < NRVEC,
                           jnp.float32(1.0), jnp.float32(0.0))
        w = yf * dv * pad_ok
        wfv[pl.ds(i * 16, 16)] = w
        mxv[...] = jnp.maximum(mxv[...], jnp.abs(w))
        return _
      lax.fori_loop(0, NVEC, y_fn, None)
      pltpu.sync_copy(mxv, mw_out.at[pl.ds(sid * 16, 16)])
    else:
      def yout_fn(i, _):
        zf = wiv[pl.ds(i * 16, 16)].astype(jnp.float32)
        tmp[pl.ds(i * 16, 16)] = zf * svec_inv * dvv[pl.ds(i * 16, 16)] + bvec
        return _
      lax.fori_loop(0, NVEC, yout_fn, None)
      pltpu.sync_copy(tmp, out_hbm.at[nsl])


@jax.jit
def _run(xpad, spad, dpad, scarr):
  mesh = plsc.VectorSubcoreMesh(
      core_axis_name="c", subcore_axis_name="s", num_cores=1)
  f = pl.kernel(
      _body,
      out_type=jax.ShapeDtypeStruct((NP,), jnp.float32),
      mesh=mesh,
      scratch_types=[
          pltpu.VMEM_SHARED((NP,), jnp.int32),
          pltpu.VMEM_SHARED((NP,), jnp.int32),
          pltpu.VMEM_SHARED((NTILES * 16,), jnp.float32),
          pltpu.VMEM_SHARED((NTILES * 16,), jnp.float32),
          pltpu.VMEM_SHARED((NTILES * 16,), jnp.int32),
          pltpu.VMEM((NS,), jnp.float32),
          pltpu.VMEM((NS,), jnp.float32),
          pltpu.VMEM((NS,), jnp.int32),
          pltpu.VMEM((NS,), jnp.float32),
          pltpu.VMEM((CH,), jnp.int32),
          pltpu.VMEM((CH,), jnp.int32),
          pltpu.VMEM((CH,), jnp.int32),
          pltpu.VMEM((DCH,), jnp.int32),
          pltpu.VMEM((DCH,), jnp.int32),
          pltpu.VMEM((13, 16), jnp.float32),
          pltpu.VMEM((NTILES * 16,), jnp.float32),
          pltpu.VMEM((NTILES * 16,), jnp.int32),
          pltpu.VMEM((16,), jnp.float32),
          pltpu.VMEM((16,), jnp.int32),
          pltpu.VMEM((16,), jnp.int32),
      ],
  )
  return f(xpad, spad, dpad, scarr)


def kernel(x, edge_index, W0, b0, Wmid, bmid, Wout, bout):
  hi = lax.Precision.HIGHEST
  t = Wout
  betas = [None] * NLAYERS
  betas[NLAYERS - 1] = bout[0]
  for i in range(L_MID, -1, -1):
    b_i = bmid[i - 1] if i >= 1 else b0
    betas[i] = jnp.matmul(b_i, t, precision=hi)[0]
    M_i = Wmid[i - 1] if i >= 1 else W0
    t = jnp.matmul(M_i, t, precision=hi)
  s_scalar = t[0, 0]

  src = edge_index[0]
  dst = edge_index[1]
  spad = jnp.concatenate([src, jnp.zeros((EP - E,), jnp.int32)])
  dpad = jnp.concatenate([dst, jnp.full((EP - E,), N, jnp.int32)])
  xpad = jnp.concatenate([x[:, 0], jnp.zeros((NP - N,), jnp.float32)])
  scarr = jnp.broadcast_to(
      jnp.stack(betas + [s_scalar])[:, None], (NLAYERS + 1, 16))

  out = _run(xpad, spad, dpad, scarr)
  return out[:N][:, None, None]


# speedup vs baseline: 108.0914x; 1.5464x over previous
import jax
import jax.numpy as jnp
from jax import lax
from jax.experimental import pallas as pl
from jax.experimental.pallas import tpu as pltpu
from jax.experimental.pallas import tpu_sc as plsc

N = 50000
E = 800000
L_MID = 10
NLAYERS = 12

NTILES = 16
NP = 50176
NS = NP // NTILES
NVEC = NS // 16
NRVEC = N // 16
EP = 819200
ET = EP // NTILES
CH = 1600
NCH = ET // CH


def _exp_vec(bits_vec):
  return ((bits_vec >> 23) & 255) - 127 + 1


def _body(x_hbm, s_hbm, d_hbm, sc_hbm, out_hbm,
          wbuf, zbuf, mwa, mwb, dmx,
          wfv, dvv, wiv, vala, valb,
          scv, mrd, mrdi, mxv, dxv, mdv,
          semg, sema, semb, *ebufs):
  s_bufs = ebufs[:NCH]
  d_bufs = ebufs[NCH:]
  sid = lax.axis_index("s")
  nb = sid * NS
  nsl = pl.ds(nb, NS)

  pltpu.sync_copy(sc_hbm, scv)
  svec = scv[12]

  for k in range(NCH):
    eb = sid * ET + k * CH
    pltpu.sync_copy(s_hbm.at[pl.ds(eb, CH)], s_bufs[k])
    pltpu.sync_copy(d_hbm.at[pl.ds(eb, CH)], d_bufs[k])

  def zero_fn(i, _):
    wiv[pl.ds(i * 16, 16)] = jnp.zeros((16,), jnp.int32)
    return _
  lax.fori_loop(0, NVEC, zero_fn, None)
  pltpu.sync_copy(wiv, zbuf.at[nsl])

  def ones_fn(i, _):
    valb[pl.ds(i * 16, 16)] = jnp.full((16,), 1, jnp.int32)
    return _
  lax.fori_loop(0, CH // 16, ones_fn, None)

  plsc.subcore_barrier()

  for k in range(NCH):
    pltpu.sync_copy(valb, zbuf.at[d_bufs[k]], add=True)

  plsc.subcore_barrier()

  pltpu.sync_copy(zbuf.at[nsl], wiv)
  dxv[...] = jnp.zeros((16,), jnp.int32)

  def rsqrt_fn(i, _):
    di = wiv[pl.ds(i * 16, 16)]
    real_ok = jnp.where(sid * NVEC + i < NRVEC, jnp.int32(1), jnp.int32(0))
    dxv[...] = jnp.maximum(dxv[...], di * real_ok)
    xv = (di + 1).astype(jnp.float32)
    ii = lax.bitcast_convert_type(xv, jnp.int32)
    ii = jnp.int32(0x5F3759DF) - lax.shift_right_logical(ii, 1)
    y = lax.bitcast_convert_type(ii, jnp.float32)
    y = y * (1.5 - 0.5 * xv * y * y)
    y = y * (1.5 - 0.5 * xv * y * y)
    y = y * (1.5 - 0.5 * xv * y * y)
    dvv[pl.ds(i * 16, 16)] = y
    return _
  lax.fori_loop(0, NVEC, rsqrt_fn, None)
  pltpu.sync_copy(dxv, dmx.at[pl.ds(sid * 16, 16)])

  pltpu.sync_copy(x_hbm.at[nsl], wfv)
  mxv[...] = jnp.zeros((16,), jnp.float32)

  def w0_fn(i, _):
    w = wfv[pl.ds(i * 16, 16)] * svec * dvv[pl.ds(i * 16, 16)]
    wfv[pl.ds(i * 16, 16)] = w
    mxv[...] = jnp.maximum(mxv[...], jnp.abs(w))
    return _
  lax.fori_loop(0, NVEC, w0_fn, None)
  pltpu.sync_copy(mxv, mwa.at[pl.ds(sid * 16, 16)])

  plsc.subcore_barrier()

  pltpu.sync_copy(dmx, mrdi)
  m = mrdi[pl.ds(0, 16)]
  for r in range(1, NTILES):
    m = jnp.maximum(m, mrdi[pl.ds(r * 16, 16)])
  mdscal = m[0]
  for l in range(1, 16):
    mdscal = jnp.maximum(mdscal, m[l])
  mdscal = mdscal + 1
  mdfv = jnp.full((16,), mdscal, jnp.int32).astype(jnp.float32)
  mdv[...] = _exp_vec(lax.bitcast_convert_type(mdfv, jnp.int32))

  for app in range(NLAYERS):
    mw_in = mwa if app % 2 == 0 else mwb
    mw_out = mwb if app % 2 == 0 else mwa
    bvec = scv[app]

    if app > 0:
      plsc.subcore_barrier()

    pltpu.sync_copy(mw_in, mrd)
    g = mrd[pl.ds(0, 16)]
    for r in range(1, NTILES):
      g = jnp.maximum(g, mrd[pl.ds(r * 16, 16)])
    gs = g[0]
    for l in range(1, 16):
      gs = jnp.maximum(gs, g[l])
    gs = jnp.maximum(gs, jnp.float32(1e-30))
    gv = jnp.full((16,), gs, jnp.float32)
    sexp = 30 - mdv[...] - _exp_vec(lax.bitcast_convert_type(gv, jnp.int32))
    svec_s = lax.bitcast_convert_type(lax.shift_left(sexp + 127, 23),
                                      jnp.float32)
    svec_inv = lax.bitcast_convert_type(lax.shift_left(127 - sexp, 23),
                                        jnp.float32)

    def cvt_fn(i, _):
      wiv[pl.ds(i * 16, 16)] = (
          wfv[pl.ds(i * 16, 16)] * svec_s).astype(jnp.int32)
      return _
    lax.fori_loop(0, NVEC, cvt_fn, None)
    pltpu.sync_copy(wiv, wbuf.at[nsl])
    pltpu.sync_copy(wiv, zbuf.at[nsl])

    plsc.subcore_barrier()

    scat = [None] * NCH
    for k in range(NCH):
      buf = vala if k % 2 == 0 else valb
      sem = sema if k % 2 == 0 else semb
      if k >= 2:
        scat[k - 2].wait()
      pltpu.async_copy(wbuf.at[s_bufs[k]], buf, semg).wait()
      scat[k] = pltpu.async_copy(buf, zbuf.at[d_bufs[k]], sem, add=True)
    scat[NCH - 2].wait()
    scat[NCH - 1].wait()

    plsc.subcore_barrier()

    pltpu.sync_copy(zbuf.at[nsl], wiv)
    if app < NLAYERS - 1:
      mxv[...] = jnp.zeros((16,), jnp.float32)

      def y_fn(i, _):
        dv = dvv[pl.ds(i * 16, 16)]
        zf = wiv[pl.ds(i * 16, 16)].astype(jnp.float32)
        yf = zf * svec_inv * dv + bvec
        pad_ok = jnp.where(sid * NVEC + i < NRVEC,
                           jnp.float32(1.0), jnp.float32(0.0))
        w = yf * dv * pad_ok
        wfv[pl.ds(i * 16, 16)] = w
        mxv[...] = jnp.maximum(mxv[...], jnp.abs(w))
        return _
      lax.fori_loop(0, NVEC, y_fn, None)
      pltpu.sync_copy(mxv, mw_out.at[pl.ds(sid * 16, 16)])
    else:
      def yout_fn(i, _):
        zf = wiv[pl.ds(i * 16, 16)].astype(jnp.float32)
        wfv[pl.ds(i * 16, 16)] = zf * svec_inv * dvv[pl.ds(i * 16, 16)] + bvec
        return _
      lax.fori_loop(0, NVEC, yout_fn, None)
      pltpu.sync_copy(wfv, out_hbm.at[nsl])


@jax.jit
def _run(xpad, spad, dpad, scarr):
  mesh = plsc.VectorSubcoreMesh(
      core_axis_name="c", subcore_axis_name="s", num_cores=1)
  f = pl.kernel(
      _body,
      out_type=jax.ShapeDtypeStruct((NP,), jnp.float32),
      mesh=mesh,
      scratch_types=[
          pltpu.VMEM_SHARED((NP,), jnp.int32),
          pltpu.VMEM_SHARED((NP,), jnp.int32),
          pltpu.VMEM_SHARED((NTILES * 16,), jnp.float32),
          pltpu.VMEM_SHARED((NTILES * 16,), jnp.float32),
          pltpu.VMEM_SHARED((NTILES * 16,), jnp.int32),
          pltpu.VMEM((NS,), jnp.float32),
          pltpu.VMEM((NS,), jnp.float32),
          pltpu.VMEM((NS,), jnp.int32),
          pltpu.VMEM((CH,), jnp.int32),
          pltpu.VMEM((CH,), jnp.int32),
          pltpu.VMEM((13, 16), jnp.float32),
          pltpu.VMEM((NTILES * 16,), jnp.float32),
          pltpu.VMEM((NTILES * 16,), jnp.int32),
          pltpu.VMEM((16,), jnp.float32),
          pltpu.VMEM((16,), jnp.int32),
          pltpu.VMEM((16,), jnp.int32),
          pltpu.SemaphoreType.DMA,
          pltpu.SemaphoreType.DMA,
          pltpu.SemaphoreType.DMA,
      ] + [pltpu.VMEM((CH,), jnp.int32)] * (2 * NCH),
  )
  return f(xpad, spad, dpad, scarr)


def kernel(x, edge_index, W0, b0, Wmid, bmid, Wout, bout):
  hi = lax.Precision.HIGHEST
  t = Wout
  betas = [None] * NLAYERS
  betas[NLAYERS - 1] = bout[0]
  for i in range(L_MID, -1, -1):
    b_i = bmid[i - 1] if i >= 1 else b0
    betas[i] = jnp.matmul(b_i, t, precision=hi)[0]
    M_i = Wmid[i - 1] if i >= 1 else W0
    t = jnp.matmul(M_i, t, precision=hi)
  s_scalar = t[0, 0]

  src = edge_index[0]
  dst = edge_index[1]
  spad = jnp.concatenate([src, jnp.zeros((EP - E,), jnp.int32)])
  dpad = jnp.concatenate([dst, jnp.full((EP - E,), N, jnp.int32)])
  xpad = jnp.concatenate([x[:, 0], jnp.zeros((NP - N,), jnp.float32)])
  scarr = jnp.broadcast_to(
      jnp.stack(betas + [s_scalar])[:, None], (NLAYERS + 1, 16))

  out = _run(xpad, spad, dpad, scarr)
  return out[:N][:, None, None]
